# baseline (speedup 1.0000x reference)
import functools

import jax
import jax.numpy as jnp
from jax import lax
from jax.experimental import pallas as pl
from jax.experimental.pallas import tpu as pltpu
from jax.experimental.pallas import tpu_sc as plsc

NC = 2
NS = 16
NW = NC * NS
L = 16

VOC = 1000000
DIM = 64
BATCH = 16384
WC = 256
RNG = 31232
TAIL_LO = 999936
TAIL_K0 = VOC - 128
OUT_ROWS = BATCH + 16
BLK_WIN = 8
BLK_ROWS = BLK_WIN * L


def _process_table(tw_hbm, tail_hbm, idx_hbm, out_hbm,
                   dbuf, idxb, hb, cb, tmpT, rows, bids, sem_in, sem_sc,
                   wid, lo, hi, nch):
    iota = lax.iota(jnp.int32, L)

    pltpu.sync_copy(idx_hbm, idxb)

    def disc(i, off):
        v = idxb[pl.ds(i * L, L)]
        m = (v >= lo) & (v < hi)
        plsc.store_compressed(hb.at[pl.ds(off, L)], i * L + iota, mask=m)
        return off + plsc.all_reduce_population_count(m)[0]

    nh = lax.fori_loop(0, BATCH // L, disc, jnp.int32(0))

    def drain_scatter(blk):
        pltpu.make_async_copy(
            out_hbm.at[pl.ds(0, BLK_ROWS)], rows.at[0], sem_sc.at[blk]).wait()

    def drain_chunk(jb):
        pltpu.make_async_copy(
            tw_hbm.at[:, pl.ds(0, WC)], dbuf.at[0], sem_in.at[jb]).wait()

    def windows(kh, k0, jb, scnt0):
        def win(hw, scnt):
            wcnt = scnt % BLK_WIN
            blk = (scnt // BLK_WIN) % 2

            @pl.when((wcnt == 0) & (scnt // BLK_WIN >= 2))
            def _():
                drain_scatter(blk)

            @pl.when(wcnt == 0)
            def _():
                def binit(q, carry):
                    bids[blk, pl.ds(q * L, L)] = jnp.full((L,), BATCH,
                                                          jnp.int32)
                    return carry
                lax.fori_loop(0, BLK_WIN, binit, 0)

            mv = (hw * L + iota) < kh
            bid16 = cb[pl.ds(hw * L, L)]
            bsafe = jnp.where(mv, bid16, 0)
            v16 = plsc.load_gather(idxb, [bsafe])
            ev = jnp.where(mv, v16 - k0, 0)
            jbv = jnp.full((L,), jb, jnp.int32)

            def cgather(c, carry):
                g = plsc.load_gather(
                    dbuf, [jbv, jnp.full((L,), c, jnp.int32), ev])
                tmpT[c, pl.ds(0, L)] = g
                return carry

            lax.fori_loop(0, DIM, cgather, 0)

            def trow(t, carry):
                tv = jnp.full((L,), t, jnp.int32)
                for k in range(DIM // L):
                    part = plsc.load_gather(tmpT, [k * L + iota, tv])
                    rows[blk, wcnt * L + t, pl.ds(k * L, L)] = part
                return carry

            lax.fori_loop(0, L, trow, 0)
            bids[blk, pl.ds(wcnt * L, L)] = jnp.where(mv, bid16, BATCH)

            @pl.when(wcnt == BLK_WIN - 1)
            def _():
                pltpu.async_copy(rows.at[blk], out_hbm.at[bids.at[blk]],
                                 sem_sc.at[blk])
            return scnt + 1

        return lax.fori_loop(0, (kh + L - 1) // L, win, scnt0)

    def rescan(vlo, vhi):
        def rs(i, kh):
            valid = (i * L + iota) < nh
            bidv = hb[pl.ds(i * L, L)]
            bsafe = jnp.where(valid, bidv, 0)
            v = plsc.load_gather(idxb, [bsafe])
            m = valid & (v >= vlo) & (v < vhi)
            plsc.store_compressed(cb.at[pl.ds(kh, L)], bidv, mask=m)
            return kh + plsc.all_reduce_population_count(m)[0]

        return lax.fori_loop(0, (nh + L - 1) // L, rs, jnp.int32(0))

    pltpu.async_copy(tw_hbm.at[:, pl.ds(pl.multiple_of(lo, 128), WC)],
                     dbuf.at[0], sem_in.at[0])

    def chunk(j, scnt):
        jb = j % 2
        k0 = pl.multiple_of(lo + j * WC, 128)

        @pl.when(j + 1 < nch)
        def _():
            k1 = pl.multiple_of(lo + (j + 1) * WC, 128)
            pltpu.async_copy(tw_hbm.at[:, pl.ds(k1, WC)],
                             dbuf.at[(j + 1) % 2], sem_in.at[(j + 1) % 2])

        drain_chunk(jb)
        kh = rescan(k0, k0 + WC)
        return windows(kh, k0, jb, scnt)

    scnt = lax.fori_loop(0, nch, chunk, jnp.int32(0))

    def tail_fn(s):
        pltpu.sync_copy(tail_hbm, dbuf.at[0, :, pl.ds(0, 128)])
        kh = rescan(TAIL_LO, VOC)
        return windows(kh, TAIL_K0, 0, s)

    scnt = lax.cond(wid == NW - 1, tail_fn, lambda s: s, scnt)

    @pl.when(scnt % BLK_WIN != 0)
    def _():
        blk = (scnt // BLK_WIN) % 2
        pltpu.async_copy(rows.at[blk], out_hbm.at[bids.at[blk]],
                         sem_sc.at[blk])

    nbs = (scnt + BLK_WIN - 1) // BLK_WIN

    @pl.when(nbs >= 1)
    def _():
        drain_scatter((nbs - 1) % 2)

    @pl.when(nbs >= 2)
    def _():
        drain_scatter((nbs - 2) % 2)


def _sc_body(twc_hbm, twx_hbm, tailc_hbm, tailx_hbm, cidx_hbm, xidx_hbm,
             ce_hbm, xe_hbm,
             dbuf, idxb, hb, cb, tmpT, rows, bids, sem_in, sem_sc):
    wid = lax.axis_index("s") * NC + lax.axis_index("c")
    lo = wid * RNG
    is_last = wid == NW - 1
    hi = jnp.where(is_last, VOC, lo + RNG)
    nch = jnp.where(is_last, 124, 122)
    _process_table(twc_hbm, tailc_hbm, cidx_hbm, ce_hbm,
                   dbuf, idxb, hb, cb, tmpT, rows, bids, sem_in, sem_sc,
                   wid, lo, hi, nch)
    _process_table(twx_hbm, tailx_hbm, xidx_hbm, xe_hbm,
                   dbuf, idxb, hb, cb, tmpT, rows, bids, sem_in, sem_sc,
                   wid, lo, hi, nch)


def _tc_body(ce_ref, xe_ref, o_ref):
    c = ce_ref[:, :DIM]
    x = xe_ref[:, :DIM]
    o_ref[0, 0, :] = jnp.sum(c * x, axis=1)


@jax.jit
def _run(cidx, xidx, cw, xw):
    twc = cw.T
    twx = xw.T
    tailc = lax.slice(twc, (0, TAIL_K0), (DIM, VOC))
    tailx = lax.slice(twx, (0, TAIL_K0), (DIM, VOC))

    mesh = plsc.VectorSubcoreMesh(
        core_axis_name="c", subcore_axis_name="s",
        num_cores=NC, num_subcores=NS)
    phase1 = pl.kernel(
        _sc_body,
        out_type=(jax.ShapeDtypeStruct((OUT_ROWS, 128), jnp.float32),
                  jax.ShapeDtypeStruct((OUT_ROWS, 128), jnp.float32)),
        mesh=mesh,
        compiler_params=pltpu.CompilerParams(
            needs_layout_passes=False, use_tc_tiling_on_sc=True),
        scratch_types=[
            pltpu.VMEM((2, DIM, WC), jnp.float32),
            pltpu.VMEM((BATCH,), jnp.int32),
            pltpu.VMEM((BATCH + L,), jnp.int32),
            pltpu.VMEM((BATCH + L,), jnp.int32),
            pltpu.VMEM((DIM, L), jnp.float32),
            pltpu.VMEM((2, BLK_ROWS, 128), jnp.float32),
            pltpu.VMEM((2, BLK_ROWS), jnp.int32),
            pltpu.SemaphoreType.DMA((2,)),
            pltpu.SemaphoreType.DMA((2,)),
        ],
    )
    ce, xe = phase1(twc, twx, tailc, tailx, cidx, xidx)

    out3 = pl.pallas_call(
        _tc_body,
        grid=(BATCH // 512,),
        in_specs=[pl.BlockSpec((512, 128), lambda i: (i, 0)),
                  pl.BlockSpec((512, 128), lambda i: (i, 0))],
        out_specs=pl.BlockSpec((1, 1, 512), lambda i: (i, 0, 0)),
        out_shape=jax.ShapeDtypeStruct((BATCH // 512, 1, 512), jnp.float32),
    )(ce, xe)
    return out3.reshape(BATCH)


def kernel(center_idx, context_idx, center_W, context_W):
    return _run(center_idx.astype(jnp.int32), context_idx.astype(jnp.int32),
                center_W, context_W)


# speedup vs baseline: 1.0027x; 1.0027x over previous
import functools

import jax
import jax.numpy as jnp
from jax import lax
from jax.experimental import pallas as pl
from jax.experimental.pallas import tpu as pltpu
from jax.experimental.pallas import tpu_sc as plsc

NC = 2
NS = 16
NW = NC * NS
L = 16

VOC = 1000000
DIM = 64
BATCH = 16384
WC = 256
RNG = 31232
TAIL_LO = 999936
TAIL_K0 = VOC - 128
OUT_ROWS = BATCH + 16
BLK_WIN = 6
BLK_ROWS = BLK_WIN * L


def _process_table(tw_hbm, tail_hbm, idx_hbm, out_hbm,
                   dbuf, idxb, hb, cb, tmpT, rows, bids, scat, scatb,
                   sem_in, sem_sc, wid, lo, hi, nch):
    iota = lax.iota(jnp.int32, L)

    pltpu.sync_copy(idx_hbm, idxb)

    def disc(i, off):
        v = idxb[pl.ds(i * L, L)]
        m = (v >= lo) & (v < hi)
        plsc.store_compressed(hb.at[pl.ds(off, L)], i * L + iota, mask=m)
        return off + plsc.all_reduce_population_count(m)[0]

    nh = lax.fori_loop(0, BATCH // L, disc, jnp.int32(0))

    def drain_scatter(blk):
        pltpu.make_async_copy(
            out_hbm.at[pl.ds(0, BLK_ROWS)], scat.at[0], sem_sc.at[blk]).wait()

    def flush(scnt, nfill):
        fidx = scnt // BLK_WIN
        blk = fidx % 2

        @pl.when(fidx >= 2)
        def _():
            drain_scatter(blk)

        def pad(q, carry):
            bids[pl.ds(q * L, L)] = jnp.full((L,), BATCH, jnp.int32)
            return carry

        lax.fori_loop(nfill, BLK_WIN, pad, 0)

        def cprow(r, carry):
            for k in range(DIM // L):
                scat[blk, r, pl.ds(k * L, L)] = rows[r, pl.ds(k * L, L)]
            return carry

        lax.fori_loop(0, BLK_ROWS, cprow, 0)

        def cpbid(q, carry):
            scatb[blk, pl.ds(q * L, L)] = bids[pl.ds(q * L, L)]
            return carry

        lax.fori_loop(0, BLK_WIN, cpbid, 0)
        pltpu.async_copy(scat.at[blk], out_hbm.at[scatb.at[blk]],
                         sem_sc.at[blk])

    def drain_chunk(jb):
        pltpu.make_async_copy(
            tw_hbm.at[:, pl.ds(0, WC)], dbuf.at[0], sem_in.at[jb]).wait()

    def windows(kh, k0, jb, scnt0):
        def win(hw, scnt):
            wcnt = scnt % BLK_WIN
            mv = (hw * L + iota) < kh
            bid16 = cb[pl.ds(hw * L, L)]
            bsafe = jnp.where(mv, bid16, 0)
            v16 = plsc.load_gather(idxb, [bsafe])
            ev = jnp.where(mv, v16 - k0, 0)
            jbv = jnp.full((L,), jb, jnp.int32)

            def cgather(c, carry):
                g = plsc.load_gather(
                    dbuf, [jbv, jnp.full((L,), c, jnp.int32), ev])
                tmpT[c, pl.ds(0, L)] = g
                return carry

            lax.fori_loop(0, DIM, cgather, 0)

            def trow(t, carry):
                tv = jnp.full((L,), t, jnp.int32)
                for k in range(DIM // L):
                    part = plsc.load_gather(tmpT, [k * L + iota, tv])
                    rows[wcnt * L + t, pl.ds(k * L, L)] = part
                return carry

            lax.fori_loop(0, L, trow, 0)
            bids[pl.ds(wcnt * L, L)] = jnp.where(mv, bid16, BATCH)

            @pl.when(wcnt == BLK_WIN - 1)
            def _():
                flush(scnt, BLK_WIN)
            return scnt + 1

        return lax.fori_loop(0, (kh + L - 1) // L, win, scnt0)

    def rescan(vlo, vhi):
        def rs(i, kh):
            valid = (i * L + iota) < nh
            bidv = hb[pl.ds(i * L, L)]
            bsafe = jnp.where(valid, bidv, 0)
            v = plsc.load_gather(idxb, [bsafe])
            m = valid & (v >= vlo) & (v < vhi)
            plsc.store_compressed(cb.at[pl.ds(kh, L)], bidv, mask=m)
            return kh + plsc.all_reduce_population_count(m)[0]

        return lax.fori_loop(0, (nh + L - 1) // L, rs, jnp.int32(0))

    pltpu.async_copy(tw_hbm.at[:, pl.ds(pl.multiple_of(lo, 128), WC)],
                     dbuf.at[0], sem_in.at[0])

    def chunk(j, scnt):
        jb = j % 2
        k0 = pl.multiple_of(lo + j * WC, 128)

        @pl.when(j + 1 < nch)
        def _():
            k1 = pl.multiple_of(lo + (j + 1) * WC, 128)
            pltpu.async_copy(tw_hbm.at[:, pl.ds(k1, WC)],
                             dbuf.at[(j + 1) % 2], sem_in.at[(j + 1) % 2])

        drain_chunk(jb)
        kh = rescan(k0, k0 + WC)
        return windows(kh, k0, jb, scnt)

    scnt = lax.fori_loop(0, nch, chunk, jnp.int32(0))

    def tail_fn(s):
        pltpu.sync_copy(tail_hbm, dbuf.at[0, :, pl.ds(0, 128)])
        kh = rescan(TAIL_LO, VOC)
        return windows(kh, TAIL_K0, 0, s)

    scnt = lax.cond(wid == NW - 1, tail_fn, lambda s: s, scnt)

    @pl.when(scnt % BLK_WIN != 0)
    def _():
        flush(scnt, scnt % BLK_WIN)

    nbs = (scnt + BLK_WIN - 1) // BLK_WIN

    @pl.when(nbs >= 1)
    def _():
        drain_scatter((nbs - 1) % 2)

    @pl.when(nbs >= 2)
    def _():
        drain_scatter((nbs - 2) % 2)


def _sc_body(twc_hbm, twx_hbm, tailc_hbm, tailx_hbm, cidx_hbm, xidx_hbm,
             ce_hbm, xe_hbm,
             dbuf, idxb, hb, cb, tmpT, rows, bids, scat, scatb,
             sem_in, sem_sc):
    wid = lax.axis_index("s") * NC + lax.axis_index("c")
    lo = wid * RNG
    is_last = wid == NW - 1
    hi = jnp.where(is_last, VOC, lo + RNG)
    nch = jnp.where(is_last, 124, 122)
    _process_table(twc_hbm, tailc_hbm, cidx_hbm, ce_hbm,
                   dbuf, idxb, hb, cb, tmpT, rows, bids, scat, scatb,
                   sem_in, sem_sc, wid, lo, hi, nch)
    _process_table(twx_hbm, tailx_hbm, xidx_hbm, xe_hbm,
                   dbuf, idxb, hb, cb, tmpT, rows, bids, scat, scatb,
                   sem_in, sem_sc, wid, lo, hi, nch)


def _tc_body(ce_ref, xe_ref, o_ref):
    c = ce_ref[:, :DIM]
    x = xe_ref[:, :DIM]
    o_ref[0, 0, :] = jnp.sum(c * x, axis=1)


@jax.jit
def _run(cidx, xidx, cw, xw):
    twc = cw.T
    twx = xw.T
    tailc = lax.slice(twc, (0, TAIL_K0), (DIM, VOC))
    tailx = lax.slice(twx, (0, TAIL_K0), (DIM, VOC))

    mesh = plsc.VectorSubcoreMesh(
        core_axis_name="c", subcore_axis_name="s",
        num_cores=NC, num_subcores=NS)
    phase1 = pl.kernel(
        _sc_body,
        out_type=(jax.ShapeDtypeStruct((OUT_ROWS, 128), jnp.float32),
                  jax.ShapeDtypeStruct((OUT_ROWS, 128), jnp.float32)),
        mesh=mesh,
        compiler_params=pltpu.CompilerParams(
            needs_layout_passes=False, use_tc_tiling_on_sc=True),
        scratch_types=[
            pltpu.VMEM((2, DIM, WC), jnp.float32),
            pltpu.VMEM((BATCH,), jnp.int32),
            pltpu.VMEM((BATCH + L,), jnp.int32),
            pltpu.VMEM((BATCH + L,), jnp.int32),
            pltpu.VMEM((DIM, L), jnp.float32),
            pltpu.VMEM((BLK_ROWS, 128), jnp.float32),
            pltpu.VMEM((BLK_ROWS,), jnp.int32),
            pltpu.VMEM((2, BLK_ROWS, 128), jnp.float32),
            pltpu.VMEM((2, BLK_ROWS), jnp.int32),
            pltpu.SemaphoreType.DMA((2,)),
            pltpu.SemaphoreType.DMA((2,)),
        ],
    )
    ce, xe = phase1(twc, twx, tailc, tailx, cidx, xidx)

    out3 = pl.pallas_call(
        _tc_body,
        grid=(BATCH // 512,),
        in_specs=[pl.BlockSpec((512, 128), lambda i: (i, 0)),
                  pl.BlockSpec((512, 128), lambda i: (i, 0))],
        out_specs=pl.BlockSpec((1, 1, 512), lambda i: (i, 0, 0)),
        out_shape=jax.ShapeDtypeStruct((BATCH // 512, 1, 512), jnp.float32),
    )(ce, xe)
    return out3.reshape(BATCH)


def kernel(center_idx, context_idx, center_W, context_W):
    return _run(center_idx.astype(jnp.int32), context_idx.astype(jnp.int32),
                center_W, context_W)


# speedup vs baseline: 3.8121x; 3.8018x over previous
import functools

import jax
import jax.numpy as jnp
from jax import lax
from jax.experimental import pallas as pl
from jax.experimental.pallas import tpu as pltpu
from jax.experimental.pallas import tpu_sc as plsc

NC = 2
NS = 16
NW = NC * NS
L = 16

BATCH = 16384
DIM = 64
B_PER_W = BATCH // NW
CHUNK = 128
NCHUNK = B_PER_W // CHUNK


def _sc_body(cidx_hbm, xidx_hbm, cw_hbm, xw_hbm, out_hbm,
             cidx_v, xidx_v, crows_v, xrows_v, out_v, sem):
    wid = lax.axis_index("s") * NC + lax.axis_index("c")
    base = wid * B_PER_W

    pltpu.sync_copy(cidx_hbm.at[pl.ds(wid * NCHUNK, NCHUNK)], cidx_v)
    pltpu.sync_copy(xidx_hbm.at[pl.ds(wid * NCHUNK, NCHUNK)], xidx_v)

    copies = []
    for j in range(NCHUNK):
        copies.append(pltpu.async_copy(
            cw_hbm.at[cidx_v.at[j]], crows_v.at[pl.ds(j * CHUNK, CHUNK)], sem))
        copies.append(pltpu.async_copy(
            xw_hbm.at[xidx_v.at[j]], xrows_v.at[pl.ds(j * CHUNK, CHUNK)], sem))
    for c in copies:
        c.wait()

    iota = lax.iota(jnp.int32, L)

    def group(g, carry):
        r0 = g * L
        vec = jnp.zeros((L,), jnp.float32)
        for u in range(L):
            r = r0 + u
            s = jnp.zeros((L,), jnp.float32)
            for k in range(DIM // L):
                cg = crows_v[r, pl.ds(k * L, L)]
                xg = xrows_v[r, pl.ds(k * L, L)]
                s = s + cg * xg
            vec = jnp.where(iota == u, jnp.sum(s), vec)
        out_v[pl.ds(r0, L)] = vec
        return carry

    lax.fori_loop(0, B_PER_W // L, group, 0)

    pltpu.sync_copy(out_v, out_hbm.at[pl.ds(base, B_PER_W)])


@functools.partial(jax.jit, static_argnames=())
def _run(cidx, xidx, cw, xw):
    mesh = plsc.VectorSubcoreMesh(
        core_axis_name="c", subcore_axis_name="s",
        num_cores=NC, num_subcores=NS)
    f = pl.kernel(
        _sc_body,
        out_type=jax.ShapeDtypeStruct((BATCH,), jnp.float32),
        mesh=mesh,
        compiler_params=pltpu.CompilerParams(
            needs_layout_passes=False, use_tc_tiling_on_sc=False),
        scratch_types=[
            pltpu.VMEM((NCHUNK, CHUNK), jnp.int32),
            pltpu.VMEM((NCHUNK, CHUNK), jnp.int32),
            pltpu.VMEM((B_PER_W, DIM), jnp.float32),
            pltpu.VMEM((B_PER_W, DIM), jnp.float32),
            pltpu.VMEM((B_PER_W,), jnp.float32),
            pltpu.SemaphoreType.DMA,
        ],
    )
    return f(cidx, xidx, cw, xw)


def kernel(center_idx, context_idx, center_W, context_W):
    cidx = center_idx.astype(jnp.int32).reshape(NW * NCHUNK, CHUNK)
    xidx = context_idx.astype(jnp.int32).reshape(NW * NCHUNK, CHUNK)
    return _run(cidx, xidx, center_W, context_W)
